# baseline (speedup 1.0000x reference)
import functools

import jax
import jax.numpy as jnp
from jax import lax
from jax.experimental import pallas as pl
from jax.experimental.pallas import tpu as pltpu
from jax.experimental.pallas import tpu_sc as plsc

B, S, H = 4, 2048, 1024
T = 2
N = B * S
NW = 32
NSLICE = 4
SSL = S // NSLICE
NSL = B * SSL
TPW = NSL // NW
WPB = NW // B
C = 32
NCHUNK = TPW // C

_mesh = plsc.VectorSubcoreMesh(core_axis_name="c", subcore_axis_name="s")

_SC_SCRATCH = [
    pltpu.VMEM((TPW,), jnp.int32),
    pltpu.VMEM((2, C, H), jnp.float32),
    pltpu.SemaphoreType.DMA,
    pltpu.SemaphoreType.DMA,
    pltpu.SemaphoreType.DMA,
    pltpu.SemaphoreType.DMA,
]


def _make_sc_gather(slice_i):
    @functools.partial(
        pl.kernel,
        mesh=_mesh,
        out_type=jax.ShapeDtypeStruct((NSL, H), jnp.float32),
        scratch_types=_SC_SCRATCH,
    )
    def _sc_gather(ids_hbm, word_hbm, out_hbm, idx_v, wbuf, g0, g1, o0, o1):
        wid = lax.axis_index("s") * 2 + lax.axis_index("c")
        b = wid // WPB
        local = (wid % WPB) * TPW
        col0 = slice_i * SSL + local
        out0 = b * SSL + local
        gsem = (g0, g1)
        osem = (o0, o1)

        pltpu.sync_copy(ids_hbm.at[b, pl.ds(col0, TPW)], idx_v)

        def start_gather(k):
            idx = idx_v.at[pl.ds(k * C, C)]
            return pltpu.async_copy(word_hbm.at[idx], wbuf.at[k % 2],
                                    gsem[k % 2])

        gcp = {k: start_gather(k) for k in range(min(2, NCHUNK))}
        ocp = {}
        for k in range(NCHUNK):
            gcp.pop(k).wait()
            ocp[k] = pltpu.async_copy(wbuf.at[k % 2],
                                      out_hbm.at[pl.ds(out0 + k * C, C)],
                                      osem[k % 2])
            if k + 2 < NCHUNK:
                ocp.pop(k).wait()
                gcp[k + 2] = start_gather(k + 2)
        for d in ocp.values():
            d.wait()

    return _sc_gather


_SC_GATHERS = [_make_sc_gather(i) for i in range(NSLICE)]


def _add_body_first(w_ref, pos_ref, typ_ref, oh_ref, out_ref):
    typed = jnp.dot(oh_ref[0], typ_ref[...],
                    preferred_element_type=jnp.float32)
    out_ref[...] = w_ref[...] + pos_ref[...] + typed


def _add_body_chain(acc_ref, w_ref, pos_ref, typ_ref, oh_ref, out_ref):
    del acc_ref
    _add_body_first(w_ref, pos_ref, typ_ref, oh_ref, out_ref)


def _tc_add_slice(i, w_i, pos, typ, oh, acc):
    nsb = S // SSL
    w_spec = pl.BlockSpec((SSL, H), lambda b: (b, 0))
    pos_spec = pl.BlockSpec((SSL, H), lambda b, _i=i: (_i, 0))
    typ_spec = pl.BlockSpec((T, H), lambda b: (0, 0))
    oh_spec = pl.BlockSpec((1, SSL, T), lambda b, _i=i: (b, _i, 0))
    out_spec = pl.BlockSpec((SSL, H), lambda b, _i=i: (b * nsb + _i, 0))
    out_shape = jax.ShapeDtypeStruct((N, H), jnp.float32)
    if acc is None:
        return pl.pallas_call(
            _add_body_first,
            grid=(B,),
            in_specs=[w_spec, pos_spec, typ_spec, oh_spec],
            out_specs=out_spec,
            out_shape=out_shape,
        )(w_i, pos, typ, oh)
    return pl.pallas_call(
        _add_body_chain,
        grid=(B,),
        in_specs=[pl.BlockSpec(memory_space=pl.ANY),
                  w_spec, pos_spec, typ_spec, oh_spec],
        out_specs=out_spec,
        out_shape=out_shape,
        input_output_aliases={0: 0},
    )(acc, w_i, pos, typ, oh)


def kernel(input_ids, token_type_ids, word_embeddings, position_embeddings,
           token_type_embeddings):
    oh = jax.nn.one_hot(token_type_ids, T, dtype=jnp.float32)
    acc = None
    for i in range(NSLICE):
        w_i = _SC_GATHERS[i](input_ids.astype(jnp.int32), word_embeddings)
        acc = _tc_add_slice(i, w_i, position_embeddings,
                            token_type_embeddings, oh, acc)
    return acc.reshape(B, S, H)


# speedup vs baseline: 1.1220x; 1.1220x over previous
import functools

import jax
import jax.numpy as jnp
from jax import lax
from jax.experimental import pallas as pl
from jax.experimental.pallas import tpu as pltpu
from jax.experimental.pallas import tpu_sc as plsc

B, S, H = 4, 2048, 1024
T = 2
N = B * S
NW = 32
TPW = N // NW
C = 32
NCHUNK = TPW // C
NBLK = 2048

_mesh = plsc.VectorSubcoreMesh(core_axis_name="c", subcore_axis_name="s")


@functools.partial(
    pl.kernel,
    mesh=_mesh,
    out_type=jax.ShapeDtypeStruct((N, H), jnp.float32),
    scratch_types=[
        pltpu.VMEM((TPW,), jnp.int32),
        pltpu.VMEM((2, C, H), jnp.float32),
        pltpu.SemaphoreType.DMA,
        pltpu.SemaphoreType.DMA,
        pltpu.SemaphoreType.DMA,
        pltpu.SemaphoreType.DMA,
    ],
)
def _sc_gather(ids_hbm, word_hbm, out_hbm, idx_v, wbuf, g0, g1, o0, o1):
    wid = lax.axis_index("s") * 2 + lax.axis_index("c")
    tok0 = wid * TPW
    b = tok0 // S
    col0 = tok0 % S
    gsem = (g0, g1)
    osem = (o0, o1)

    pltpu.sync_copy(ids_hbm.at[b, pl.ds(col0, TPW)], idx_v)

    def start_gather(k):
        idx = idx_v.at[pl.ds(k * C, C)]
        return pltpu.async_copy(word_hbm.at[idx], wbuf.at[k % 2], gsem[k % 2])

    gcp = {0: start_gather(0), 1: start_gather(1)}
    ocp = {}
    for k in range(NCHUNK):
        gcp.pop(k).wait()
        ocp[k] = pltpu.async_copy(wbuf.at[k % 2],
                                  out_hbm.at[pl.ds(tok0 + k * C, C)],
                                  osem[k % 2])
        if k + 2 < NCHUNK:
            ocp.pop(k).wait()
            gcp[k + 2] = start_gather(k + 2)
    for d in ocp.values():
        d.wait()


def _add_body(w_ref, pos_ref, typ_ref, oh_ref, out_ref):
    m = oh_ref[0, :, 0:1]
    typed = jnp.where(m > 0.5, typ_ref[0:1, :], typ_ref[1:2, :])
    out_ref[...] = w_ref[...] + pos_ref[...] + typed


def _tc_add(w, pos, typ, oh):
    nj = S // NBLK
    return pl.pallas_call(
        _add_body,
        grid=(nj, B),
        in_specs=[
            pl.BlockSpec((NBLK, H), lambda j, b: (b * nj + j, 0)),
            pl.BlockSpec((NBLK, H), lambda j, b: (j, 0)),
            pl.BlockSpec((T, H), lambda j, b: (0, 0)),
            pl.BlockSpec((1, NBLK, T), lambda j, b: (b, j, 0)),
        ],
        out_specs=pl.BlockSpec((NBLK, H), lambda j, b: (b * nj + j, 0)),
        out_shape=jax.ShapeDtypeStruct((N, H), jnp.float32),
    )(w, pos, typ, oh)


def kernel(input_ids, token_type_ids, word_embeddings, position_embeddings,
           token_type_embeddings):
    oh = jax.nn.one_hot(token_type_ids, T, dtype=jnp.float32)
    w = _sc_gather(input_ids.astype(jnp.int32), word_embeddings)
    out = _tc_add(w, position_embeddings, token_type_embeddings, oh)
    return out.reshape(B, S, H)
